# baseline (speedup 1.0000x reference)
import jax
import jax.numpy as jnp
from jax import lax
from jax.experimental import pallas as pl
from jax.experimental.pallas import tpu as pltpu
from jax.experimental.pallas import tpu_sc as plsc

_B, _N, _E, _F, _H, _C = 4, 10000, 160000, 256, 3, 128
_HC = _H * _C
_TN = _B * _N
_NE = _B * _E
_NC, _NS = 2, 16
_NW = _NC * _NS
_EPW = _NE // _NW
_T = 2000
_NT = _EPW // _T
_BS = 32
_NSWEEP = 8
_WOWN = 160
_SWR = _NW * _WOWN
_TNPAD = _SWR * _NSWEEP
_ACCW = 512
_RCAP = 22016
_SENT = 1 << 29
_SLOPE = 0.2


def _mm_body(x_ref, wl_ref, wr_ref, xl_ref, xr_ref):
    xb = x_ref[...]
    xl_ref[...] = jnp.dot(xb, wl_ref[...], preferred_element_type=jnp.float32)
    xr_ref[...] = jnp.dot(xb, wr_ref[...], preferred_element_type=jnp.float32)


def _matmuls(xf, W_l, W_r):
    bm = 1000
    return pl.pallas_call(
        _mm_body,
        grid=(_TN // bm,),
        in_specs=[
            pl.BlockSpec((bm, _F), lambda i: (i, 0)),
            pl.BlockSpec((_F, _HC), lambda i: (0, 0)),
            pl.BlockSpec((_F, _HC), lambda i: (0, 0)),
        ],
        out_specs=[
            pl.BlockSpec((bm, _HC), lambda i: (i, 0)),
            pl.BlockSpec((bm, _HC), lambda i: (i, 0)),
        ],
        out_shape=[jax.ShapeDtypeStruct((_TN, _HC), jnp.float32)] * 2,
    )(xf, W_l, W_r)


def _post_body(acc_ref, xl_ref, xr_ref, att_ref, bias_ref, out_ref):
    xl = xl_ref[...]
    xr = xr_ref[...]
    z = xl + xr
    e = jnp.where(z > 0, z, _SLOPE * z) * att_ref[...]
    for h in range(_H):
        sl = slice(h * _C, (h + 1) * _C)
        a = jnp.sum(e[:, sl], axis=1, keepdims=True)
        es = jnp.exp(a)
        num = acc_ref[:, sl] + es * xl[:, sl]
        den = acc_ref[:, _HC + h:_HC + h + 1] + es + 1e-16
        out_ref[:, sl] = num / den + bias_ref[:, sl]


def _post(acc, xl, xr, attf, biasf):
    bm = 1000
    return pl.pallas_call(
        _post_body,
        grid=(_TN // bm,),
        in_specs=[
            pl.BlockSpec((bm, _ACCW), lambda i: (i, 0)),
            pl.BlockSpec((bm, _HC), lambda i: (i, 0)),
            pl.BlockSpec((bm, _HC), lambda i: (i, 0)),
            pl.BlockSpec((1, _HC), lambda i: (0, 0)),
            pl.BlockSpec((1, _HC), lambda i: (0, 0)),
        ],
        out_specs=pl.BlockSpec((bm, _HC), lambda i: (i, 0)),
        out_shape=jax.ShapeDtypeStruct((_TN, _HC), jnp.float32),
    )(acc, xl, xr, attf, biasf)


_SC_MESH = plsc.VectorSubcoreMesh(core_axis_name="c", subcore_axis_name="s",
                                  num_cores=_NC, num_subcores=_NS)
_SC_CP = pltpu.CompilerParams(needs_layout_passes=False,
                              use_tc_tiling_on_sc=False)


def _bucket_body(src, dst, bsrc, bdst, counts,
                 sbuf, dbuf, lsrc, ldst, cvec):
    c = lax.axis_index("c")
    s = lax.axis_index("s")
    wid = s * _NC + c
    base_e = wid * _EPW
    iot = lax.iota(jnp.int32, 16)
    sent = jnp.full((16,), _SENT, jnp.int32)

    def tile_body(t, cursors):
        tb = pl.multiple_of(base_e + t * _T, 8)
        pltpu.sync_copy(src.at[pl.ds(tb, _T)], sbuf)
        pltpu.sync_copy(dst.at[pl.ds(tb, _T)], dbuf)

        def cpct(k, cnts):
            dv = dbuf[pl.ds(k * 16, 16)]
            sv = sbuf[pl.ds(k * 16, 16)]
            bid = dv // _SWR
            out = []
            for kk in range(_NSWEEP):
                m = bid == kk
                mi = jnp.where(m, 1, 0).astype(jnp.int32)
                pc = plsc.cumsum(mi)
                pos = cnts[kk] + pc - 1
                row = jnp.full((16,), kk, jnp.int32)
                plsc.store_scatter(ldst, [row, pos], dv, mask=m)
                plsc.store_scatter(lsrc, [row, pos], sv, mask=m)
                out.append(cnts[kk] + jnp.sum(mi))
            return tuple(out)

        cnts = lax.fori_loop(0, _T // 16, cpct, (jnp.int32(0),) * _NSWEEP)

        new_cursors = []
        for kk in range(_NSWEEP):
            cnt = cnts[kk]
            pad = (-cnt) % 8
            row = jnp.full((16,), kk, jnp.int32)
            plsc.store_scatter(ldst, [row, cnt + iot], sent, mask=iot < pad)
            rbase = (wid * _NSWEEP + kk) * _RCAP
            cura = pl.multiple_of(rbase + cursors[kk], 8)
            pltpu.sync_copy(lsrc.at[kk], bsrc.at[pl.ds(cura, _T + 16)])
            pltpu.sync_copy(ldst.at[kk], bdst.at[pl.ds(cura, _T + 16)])
            new_cursors.append(cursors[kk] + cnt + pad)
        return tuple(new_cursors)

    cursors = lax.fori_loop(0, _NT, tile_body, (jnp.int32(0),) * _NSWEEP)
    cv = jnp.zeros((16,), jnp.int32)
    for kk in range(_NSWEEP):
        cv = cv + cursors[kk] * jnp.where(iot == kk, 1, 0).astype(jnp.int32)
    cvec[...] = cv
    pltpu.sync_copy(cvec, counts.at[wid])


_bucket_call = pl.kernel(
    _bucket_body,
    out_type=(
        jax.ShapeDtypeStruct((_NW * _NSWEEP * _RCAP,), jnp.int32),
        jax.ShapeDtypeStruct((_NW * _NSWEEP * _RCAP,), jnp.int32),
        jax.ShapeDtypeStruct((_NW, 16), jnp.int32),
    ),
    mesh=_SC_MESH,
    scratch_types=[
        pltpu.VMEM((_T,), jnp.int32),
        pltpu.VMEM((_T,), jnp.int32),
        pltpu.VMEM((_NSWEEP, _T + 16), jnp.int32),
        pltpu.VMEM((_NSWEEP, _T + 16), jnp.int32),
        pltpu.VMEM((16,), jnp.int32),
    ],
    compiler_params=_SC_CP,
)


def _sweep_body(xl, xr, bsrc_in, bdst_in, counts, att6, att4, zrows, acc,
                sbuf, dbuf, cls, cld, sixv, dixv, dlv, valf,
                ubuf, vbuf, av, a6v, a4v, cnv, sem):
    c = lax.axis_index("c")
    s = lax.axis_index("s")
    wid = s * _NC + c

    pltpu.sync_copy(att6, a6v)
    pltpu.sync_copy(att4, a4v)
    pltpu.sync_copy(counts, cnv)
    iot = lax.iota(jnp.int32, 16)

    def sweep(sw, carry):
        lo = sw * _SWR + wid * _WOWN
        hi = lo + _WOWN
        swoh = jnp.where(iot == sw, 1, 0).astype(jnp.int32)
        pltpu.sync_copy(zrows, av)

        def region(r, carry_r):
            cnt_r = jnp.sum(cnv[r, pl.ds(0, 16)] * swoh)
            rbase = (r * _NSWEEP + sw) * _RCAP
            nch = (cnt_r + _T - 1) // _T

            def chunk(ci, carry_c):
                cb = pl.multiple_of(rbase + ci * _T, 8)
                pltpu.sync_copy(bsrc_in.at[pl.ds(cb, _T)], sbuf)
                pltpu.sync_copy(bdst_in.at[pl.ds(cb, _T)], dbuf)
                gbase = ci * _T

                def cpct(k, cnt):
                    dv = dbuf[pl.ds(k * 16, 16)]
                    sv = sbuf[pl.ds(k * 16, 16)]
                    live = (gbase + k * 16 + iot) < cnt_r
                    m = (dv >= lo) & (dv < hi) & live
                    mi = jnp.where(m, 1, 0).astype(jnp.int32)
                    pc = plsc.cumsum(mi)
                    pos = cnt + pc - 1
                    plsc.store_scatter(cld, [pos], dv, mask=m)
                    plsc.store_scatter(cls, [pos], sv, mask=m)
                    return cnt + jnp.sum(mi)

                cnt = lax.fori_loop(0, _T // 16, cpct, jnp.int32(0))
                nsb = (cnt + _BS - 1) // _BS

                oh = [jnp.where(iot == h, 1.0, 0.0) for h in range(_H)]

                def sub(sb, carry_s):
                    o = sb * _BS
                    for k in range(_BS // 16):
                        valid = (o + k * 16 + iot) < cnt
                        dg = cld[pl.ds(o + k * 16, 16)]
                        sg = cls[pl.ds(o + k * 16, 16)]
                        dixv[pl.ds(k * 16, 16)] = jnp.where(valid, dg, 0)
                        dlv[pl.ds(k * 16, 16)] = jnp.where(valid, dg - lo, 0)
                        sixv[pl.ds(k * 16, 16)] = jnp.where(valid, sg, 0)
                        valf[pl.ds(k * 16, 16)] = jnp.where(valid, 1.0, 0.0)
                    cp1 = pltpu.async_copy(xr.at[dixv], ubuf, sem)
                    cp2 = pltpu.async_copy(xl.at[sixv], vbuf, sem)
                    cp1.wait()
                    cp2.wait()

                    def edge(i, carry_e):
                        g16 = pl.multiple_of((i // 16) * 16, 8)
                        lidx = jnp.zeros((16,), jnp.int32) + (i % 16)
                        pib = "promise_in_bounds"
                        dspl = dlv[pl.ds(g16, 16)].at[lidx].get(mode=pib)
                        vspl = valf[pl.ds(g16, 16)].at[lidx].get(mode=pib)
                        ws = []
                        for h in range(_H):
                            ah = None
                            for j in range(8):
                                off = h * _C + j * 16
                                z = ubuf[i, pl.ds(off, 16)] \
                                    + vbuf[i, pl.ds(off, 16)]
                                t = z * a6v[pl.ds(off, 16)] \
                                    + jnp.abs(z) * a4v[pl.ds(off, 16)]
                                ah = t if ah is None else ah + t
                            t1 = ah + ah.at[iot ^ 1].get(mode=pib)
                            t2 = t1 + t1.at[iot ^ 2].get(mode=pib)
                            t4 = t2 + t2.at[iot ^ 4].get(mode=pib)
                            t8 = t4 + t4.at[iot ^ 8].get(mode=pib)
                            ws.append(jnp.exp(t8) * vspl)
                        base = dspl * _ACCW
                        for h in range(_H):
                            for j in range(8):
                                off = h * _C + j * 16
                                addr = base + (iot + off)
                                cur = plsc.load_gather(av, [addr])
                                plsc.store_scatter(
                                    av, [addr],
                                    cur + vbuf[i, pl.ds(off, 16)] * ws[h])
                        addr = base + (iot + _HC)
                        cur = plsc.load_gather(av, [addr])
                        tail = oh[0] * ws[0] + oh[1] * ws[1] + oh[2] * ws[2]
                        plsc.store_scatter(av, [addr], cur + tail)
                        return carry_e

                    lax.fori_loop(0, _BS, edge, 0)
                    return carry_s

                lax.fori_loop(0, nsb, sub, 0)
                return carry_c

            lax.fori_loop(0, nch, chunk, 0)
            return carry_r

        lax.fori_loop(0, _NW, region, 0)
        dro = pl.multiple_of(lo * _ACCW, 8)
        pltpu.sync_copy(av, acc.at[pl.ds(dro, _WOWN * _ACCW)])
        return carry

    lax.fori_loop(0, _NSWEEP, sweep, 0)


def _make_sweep_call():
    return pl.kernel(
        _sweep_body,
        out_type=jax.ShapeDtypeStruct((_TNPAD * _ACCW,), jnp.float32),
        mesh=_SC_MESH,
        scratch_types=[
            pltpu.VMEM((_T,), jnp.int32),
            pltpu.VMEM((_T,), jnp.int32),
            pltpu.VMEM((_T + 16,), jnp.int32),
            pltpu.VMEM((_T + 16,), jnp.int32),
            pltpu.VMEM((_BS,), jnp.int32),
            pltpu.VMEM((_BS,), jnp.int32),
            pltpu.VMEM((_BS,), jnp.int32),
            pltpu.VMEM((_BS,), jnp.float32),
            pltpu.VMEM((_BS, _HC), jnp.float32),
            pltpu.VMEM((_BS, _HC), jnp.float32),
            pltpu.VMEM((_WOWN * _ACCW,), jnp.float32),
            pltpu.VMEM((_HC,), jnp.float32),
            pltpu.VMEM((_HC,), jnp.float32),
            pltpu.VMEM((_NW, 16), jnp.int32),
            pltpu.SemaphoreType.DMA,
        ],
        compiler_params=_SC_CP,
    )


_sweep_call = _make_sweep_call()


def kernel(x, edge_index, W_l, W_r, att, bias):
    xf = x.reshape(_TN, _F)
    offs = (jnp.arange(_B) * _N).astype(jnp.int32)
    src = (edge_index[0][None, :].astype(jnp.int32) + offs[:, None]).reshape(-1)
    dst = (edge_index[1][None, :].astype(jnp.int32) + offs[:, None]).reshape(-1)
    attf = att.reshape(1, _HC)
    att6 = ((1.0 + _SLOPE) / 2.0 * att).reshape(_HC)
    att4 = ((1.0 - _SLOPE) / 2.0 * att).reshape(_HC)
    zrows = jnp.zeros((_WOWN * _ACCW,), jnp.float32)

    xl, xr = _matmuls(xf, W_l, W_r)
    bsrc, bdst, counts = _bucket_call(src, dst)
    acc = _sweep_call(xl, xr, bsrc, bdst, counts, att6, att4, zrows)
    acc = acc.reshape(_TNPAD, _ACCW)[:_TN]
    out = _post(acc, xl, xr, attf, bias.reshape(1, _HC))
    return out.reshape(_B, _N, _HC)


# speedup vs baseline: 1.3045x; 1.3045x over previous
import jax
import jax.numpy as jnp
from jax import lax
from jax.experimental import pallas as pl
from jax.experimental.pallas import tpu as pltpu
from jax.experimental.pallas import tpu_sc as plsc

_B, _N, _E, _F, _H, _C = 4, 10000, 160000, 256, 3, 128
_HC = _H * _C
_TN = _B * _N
_NE = _B * _E
_NC, _NS = 2, 16
_NW = _NC * _NS
_EPW = _NE // _NW
_T = 2000
_TB = 1000
_NTB = _EPW // _TB
_NT = _EPW // _T
_BS = 48
_NBK = 32
_BKR = 40960 // _NBK
_NSWEEP = 8
_WOWN = 160
_SWR = _NW * _WOWN
_TNPAD = _SWR * _NSWEEP
_ACCW = 512
_RCAP = 22016
_SENT = 1 << 29
_SLOPE = 0.2


def _mm_body(x_ref, wl_ref, wr_ref, xl_ref, xr_ref):
    xb = x_ref[...]
    xl_ref[...] = jnp.dot(xb, wl_ref[...], preferred_element_type=jnp.float32)
    xr_ref[...] = jnp.dot(xb, wr_ref[...], preferred_element_type=jnp.float32)


def _matmuls(xf, W_l, W_r):
    bm = 1000
    return pl.pallas_call(
        _mm_body,
        grid=(_TN // bm,),
        in_specs=[
            pl.BlockSpec((bm, _F), lambda i: (i, 0)),
            pl.BlockSpec((_F, _HC), lambda i: (0, 0)),
            pl.BlockSpec((_F, _HC), lambda i: (0, 0)),
        ],
        out_specs=[
            pl.BlockSpec((bm, _HC), lambda i: (i, 0)),
            pl.BlockSpec((bm, _HC), lambda i: (i, 0)),
        ],
        out_shape=[jax.ShapeDtypeStruct((_TN, _HC), jnp.float32)] * 2,
    )(xf, W_l, W_r)


def _post_body(acc_ref, xl_ref, xr_ref, att_ref, bias_ref, out_ref):
    xl = xl_ref[...]
    xr = xr_ref[...]
    z = xl + xr
    e = jnp.where(z > 0, z, _SLOPE * z) * att_ref[...]
    for h in range(_H):
        sl = slice(h * _C, (h + 1) * _C)
        a = jnp.sum(e[:, sl], axis=1, keepdims=True)
        es = jnp.exp(a)
        num = acc_ref[:, sl] + es * xl[:, sl]
        den = acc_ref[:, _HC + h:_HC + h + 1] + es + 1e-16
        out_ref[:, sl] = num / den + bias_ref[:, sl]


def _post(acc, xl, xr, attf, biasf):
    bm = 1000
    return pl.pallas_call(
        _post_body,
        grid=(_TN // bm,),
        in_specs=[
            pl.BlockSpec((bm, _ACCW), lambda i: (i, 0)),
            pl.BlockSpec((bm, _HC), lambda i: (i, 0)),
            pl.BlockSpec((bm, _HC), lambda i: (i, 0)),
            pl.BlockSpec((1, _HC), lambda i: (0, 0)),
            pl.BlockSpec((1, _HC), lambda i: (0, 0)),
        ],
        out_specs=pl.BlockSpec((bm, _HC), lambda i: (i, 0)),
        out_shape=jax.ShapeDtypeStruct((_TN, _HC), jnp.float32),
    )(acc, xl, xr, attf, biasf)


_SC_MESH = plsc.VectorSubcoreMesh(core_axis_name="c", subcore_axis_name="s",
                                  num_cores=_NC, num_subcores=_NS)
_SC_CP = pltpu.CompilerParams(needs_layout_passes=False,
                              use_tc_tiling_on_sc=False)


def _bucket_body(src, dst, bsrc, bdst, counts,
                 sbuf, dbuf, lsrc, ldst, cvec):
    c = lax.axis_index("c")
    s = lax.axis_index("s")
    wid = s * _NC + c
    base_e = wid * _EPW
    iot = lax.iota(jnp.int32, 16)
    sent = jnp.full((16,), _SENT, jnp.int32)

    def tile_body(t, cursors):
        tb = pl.multiple_of(base_e + t * _TB, 8)
        pltpu.sync_copy(src.at[pl.ds(tb, _TB)], sbuf)
        pltpu.sync_copy(dst.at[pl.ds(tb, _TB)], dbuf)

        def cpct(k, cnts):
            dv = dbuf[pl.ds(k * 16, 16)]
            sv = sbuf[pl.ds(k * 16, 16)]
            bid = dv // _BKR
            out = []
            for kk in range(_NBK):
                m = bid == kk
                mi = jnp.where(m, 1, 0).astype(jnp.int32)
                pc = plsc.cumsum(mi)
                pos = cnts[kk] + pc - 1
                row = jnp.full((16,), kk, jnp.int32)
                plsc.store_scatter(ldst, [row, pos], dv, mask=m)
                plsc.store_scatter(lsrc, [row, pos], sv, mask=m)
                out.append(cnts[kk] + jnp.sum(mi))
            return tuple(out)

        cnts = lax.fori_loop(0, _TB // 16, cpct, (jnp.int32(0),) * _NBK)

        new_cursors = []
        for kk in range(_NBK):
            cnt = cnts[kk]
            pad = (-cnt) % 8
            row = jnp.full((16,), kk, jnp.int32)
            plsc.store_scatter(ldst, [row, cnt + iot], sent, mask=iot < pad)
            rbase = (wid * _NBK + kk) * _RCAP
            cura = pl.multiple_of(rbase + cursors[kk], 8)
            pltpu.sync_copy(lsrc.at[kk], bsrc.at[pl.ds(cura, _TB + 16)])
            pltpu.sync_copy(ldst.at[kk], bdst.at[pl.ds(cura, _TB + 16)])
            new_cursors.append(cursors[kk] + cnt + pad)
        return tuple(new_cursors)

    cursors = lax.fori_loop(0, _NTB, tile_body, (jnp.int32(0),) * _NBK)
    for half in range(2):
        cv = jnp.zeros((16,), jnp.int32)
        for kk in range(16):
            cv = cv + cursors[half * 16 + kk] \
                * jnp.where(iot == kk, 1, 0).astype(jnp.int32)
        cvec[...] = cv
        pltpu.sync_copy(cvec, counts.at[wid * 2 + half])


_bucket_call = pl.kernel(
    _bucket_body,
    out_type=(
        jax.ShapeDtypeStruct((_NW * _NBK * _RCAP,), jnp.int32),
        jax.ShapeDtypeStruct((_NW * _NBK * _RCAP,), jnp.int32),
        jax.ShapeDtypeStruct((_NW * 2, 16), jnp.int32),
    ),
    mesh=_SC_MESH,
    scratch_types=[
        pltpu.VMEM((_TB,), jnp.int32),
        pltpu.VMEM((_TB,), jnp.int32),
        pltpu.VMEM((_NBK, _TB + 16), jnp.int32),
        pltpu.VMEM((_NBK, _TB + 16), jnp.int32),
        pltpu.VMEM((16,), jnp.int32),
    ],
    compiler_params=_SC_CP,
)


def _sweep_body(xl, xr, bsrc_in, bdst_in, counts, att6, att4, zrows, acc,
                sbuf, dbuf, cls, cld, sixv, dixv, dlv, valf,
                ubuf, vbuf, av, a6v, a4v, cnv, sem):
    c = lax.axis_index("c")
    s = lax.axis_index("s")
    wid = s * _NC + c

    pltpu.sync_copy(att6, a6v)
    pltpu.sync_copy(att4, a4v)
    pltpu.sync_copy(counts, cnv)
    iot = lax.iota(jnp.int32, 16)

    def sweep(sw, carry):
        lo = sw * _SWR + wid * _WOWN
        hi = lo + _WOWN
        bk = sw * 4 + wid // 8
        bkoh = jnp.where(iot == bk % 16, 1, 0).astype(jnp.int32)
        pltpu.sync_copy(zrows, av)

        def region(r, carry_r):
            cnt_r = jnp.sum(cnv[r * 2 + bk // 16, pl.ds(0, 16)] * bkoh)
            rbase = (r * _NBK + bk) * _RCAP
            nch = (cnt_r + _T - 1) // _T

            def chunk(ci, carry_c):
                cb = pl.multiple_of(rbase + ci * _T, 8)
                pltpu.sync_copy(bsrc_in.at[pl.ds(cb, _T)], sbuf)
                pltpu.sync_copy(bdst_in.at[pl.ds(cb, _T)], dbuf)
                gbase = ci * _T

                def cpct(k, cnt):
                    dv = dbuf[pl.ds(k * 16, 16)]
                    sv = sbuf[pl.ds(k * 16, 16)]
                    live = (gbase + k * 16 + iot) < cnt_r
                    m = (dv >= lo) & (dv < hi) & live
                    mi = jnp.where(m, 1, 0).astype(jnp.int32)
                    pc = plsc.cumsum(mi)
                    pos = cnt + pc - 1
                    plsc.store_scatter(cld, [pos], dv, mask=m)
                    plsc.store_scatter(cls, [pos], sv, mask=m)
                    return cnt + jnp.sum(mi)

                cnt = lax.fori_loop(0, _T // 16, cpct, jnp.int32(0))
                nsb = (cnt + _BS - 1) // _BS

                oh = [jnp.where(iot == h, 1.0, 0.0) for h in range(_H)]

                def sub(sb, carry_s):
                    o = sb * _BS
                    for k in range(_BS // 16):
                        valid = (o + k * 16 + iot) < cnt
                        dg = cld[pl.ds(o + k * 16, 16)]
                        sg = cls[pl.ds(o + k * 16, 16)]
                        dixv[pl.ds(k * 16, 16)] = jnp.where(valid, dg, 0)
                        dlv[pl.ds(k * 16, 16)] = jnp.where(valid, dg - lo, 0)
                        sixv[pl.ds(k * 16, 16)] = jnp.where(valid, sg, 0)
                        valf[pl.ds(k * 16, 16)] = jnp.where(valid, 1.0, 0.0)
                    cp1 = pltpu.async_copy(xr.at[dixv], ubuf, sem)
                    cp2 = pltpu.async_copy(xl.at[sixv], vbuf, sem)
                    cp1.wait()
                    cp2.wait()

                    def edge(i, carry_e):
                        g16 = pl.multiple_of((i // 16) * 16, 8)
                        lidx = jnp.zeros((16,), jnp.int32) + (i % 16)
                        pib = "promise_in_bounds"
                        dspl = dlv[pl.ds(g16, 16)].at[lidx].get(mode=pib)
                        vspl = valf[pl.ds(g16, 16)].at[lidx].get(mode=pib)
                        ws = []
                        for h in range(_H):
                            ah = None
                            for j in range(8):
                                off = h * _C + j * 16
                                z = ubuf[i, pl.ds(off, 16)] \
                                    + vbuf[i, pl.ds(off, 16)]
                                t = z * a6v[pl.ds(off, 16)] \
                                    + jnp.abs(z) * a4v[pl.ds(off, 16)]
                                ah = t if ah is None else ah + t
                            t1 = ah + ah.at[iot ^ 1].get(mode=pib)
                            t2 = t1 + t1.at[iot ^ 2].get(mode=pib)
                            t4 = t2 + t2.at[iot ^ 4].get(mode=pib)
                            t8 = t4 + t4.at[iot ^ 8].get(mode=pib)
                            ws.append(jnp.exp(t8) * vspl)
                        base = dspl * _ACCW
                        for h in range(_H):
                            for j in range(8):
                                off = h * _C + j * 16
                                addr = base + (iot + off)
                                cur = plsc.load_gather(av, [addr])
                                plsc.store_scatter(
                                    av, [addr],
                                    cur + vbuf[i, pl.ds(off, 16)] * ws[h])
                        addr = base + (iot + _HC)
                        cur = plsc.load_gather(av, [addr])
                        tail = oh[0] * ws[0] + oh[1] * ws[1] + oh[2] * ws[2]
                        plsc.store_scatter(av, [addr], cur + tail)
                        return carry_e

                    lax.fori_loop(0, _BS, edge, 0)
                    return carry_s

                lax.fori_loop(0, nsb, sub, 0)
                return carry_c

            lax.fori_loop(0, nch, chunk, 0)
            return carry_r

        lax.fori_loop(0, _NW, region, 0)
        dro = pl.multiple_of(lo * _ACCW, 8)
        pltpu.sync_copy(av, acc.at[pl.ds(dro, _WOWN * _ACCW)])
        return carry

    lax.fori_loop(0, _NSWEEP, sweep, 0)


def _make_sweep_call():
    return pl.kernel(
        _sweep_body,
        out_type=jax.ShapeDtypeStruct((_TNPAD * _ACCW,), jnp.float32),
        mesh=_SC_MESH,
        scratch_types=[
            pltpu.VMEM((_T,), jnp.int32),
            pltpu.VMEM((_T,), jnp.int32),
            pltpu.VMEM((_T + 16,), jnp.int32),
            pltpu.VMEM((_T + 16,), jnp.int32),
            pltpu.VMEM((_BS,), jnp.int32),
            pltpu.VMEM((_BS,), jnp.int32),
            pltpu.VMEM((_BS,), jnp.int32),
            pltpu.VMEM((_BS,), jnp.float32),
            pltpu.VMEM((_BS, _HC), jnp.float32),
            pltpu.VMEM((_BS, _HC), jnp.float32),
            pltpu.VMEM((_WOWN * _ACCW,), jnp.float32),
            pltpu.VMEM((_HC,), jnp.float32),
            pltpu.VMEM((_HC,), jnp.float32),
            pltpu.VMEM((_NW * 2, 16), jnp.int32),
            pltpu.SemaphoreType.DMA,
        ],
        compiler_params=_SC_CP,
    )


_sweep_call = _make_sweep_call()


def kernel(x, edge_index, W_l, W_r, att, bias):
    xf = x.reshape(_TN, _F)
    offs = (jnp.arange(_B) * _N).astype(jnp.int32)
    src = (edge_index[0][None, :].astype(jnp.int32) + offs[:, None]).reshape(-1)
    dst = (edge_index[1][None, :].astype(jnp.int32) + offs[:, None]).reshape(-1)
    attf = att.reshape(1, _HC)
    att6 = ((1.0 + _SLOPE) / 2.0 * att).reshape(_HC)
    att4 = ((1.0 - _SLOPE) / 2.0 * att).reshape(_HC)
    zrows = jnp.zeros((_WOWN * _ACCW,), jnp.float32)

    xl, xr = _matmuls(xf, W_l, W_r)
    bsrc, bdst, counts = _bucket_call(src, dst)
    acc = _sweep_call(xl, xr, bsrc, bdst, counts, att6, att4, zrows)
    acc = acc.reshape(_TNPAD, _ACCW)[:_TN]
    out = _post(acc, xl, xr, attf, bias.reshape(1, _HC))
    return out.reshape(_B, _N, _HC)
